# baseline (speedup 1.0000x reference)
import functools

import jax
import jax.numpy as jnp
from jax import lax
from jax.experimental import pallas as pl
from jax.experimental.pallas import tpu as pltpu
from jax.experimental.pallas import tpu_sc as plsc

B = 4096
L = 200
EMB = 64
NC = 2
NS = 16
NW = NC * NS
RPW = B // NW
SEQS = 2
ROWS = SEQS * L
NSUP = RPW // SEQS


NBUF = 2


def _make_sc_pool():
    mesh = plsc.VectorSubcoreMesh(core_axis_name="c", subcore_axis_name="s")

    @functools.partial(
        pl.kernel,
        out_type=jax.ShapeDtypeStruct((B, EMB), jnp.float32),
        mesh=mesh,
        compiler_params=pltpu.CompilerParams(use_tc_tiling_on_sc=False),
        scratch_types=[
            pltpu.VMEM((RPW * L,), jnp.int32),
            pltpu.VMEM((NBUF, ROWS, EMB), jnp.float32),
            pltpu.VMEM((RPW, EMB), jnp.float32),
        ] + [pltpu.SemaphoreType.DMA] * (NBUF + 1),
    )
    def sc_pool(idx_hbm, table_hbm, out_hbm, idx_v, bufs, pooled_v, *sems):
        sems_g = sems[:NBUF]
        sem_i = sems[NBUF]
        cid = lax.axis_index("c")
        sid = lax.axis_index("s")
        wid = sid * NC + cid
        base = wid * RPW
        zero = jnp.zeros((16,), jnp.float32)

        def stage_body(r, carry):
            pltpu.async_copy(idx_hbm.at[base + r], idx_v.at[pl.ds(r * L, L)],
                             sem_i)
            return carry

        lax.fori_loop(0, RPW, stage_body, 0)

        def drain_body(r, carry):
            pltpu.make_async_copy(idx_hbm.at[base], idx_v.at[pl.ds(0, L)],
                                  sem_i).wait()
            return carry

        lax.fori_loop(0, RPW, drain_body, 0)

        for nb in range(NBUF):
            pltpu.async_copy(table_hbm.at[idx_v.at[pl.ds(nb * ROWS, ROWS)]],
                             bufs.at[nb], sems_g[nb])

        def outer_body(g, carry):
            for nb in range(NBUF):
                s = NBUF * g + nb
                pltpu.make_async_copy(
                    table_hbm.at[idx_v.at[pl.ds(nb * ROWS, ROWS)]],
                    bufs.at[nb], sems_g[nb]).wait()
                for t in range(SEQS):
                    acc = (zero,) * (EMB // 16)

                    def acc_body(i, accs, nb=nb, t=t):
                        a = list(accs)
                        for u in range(4):
                            row = t * L + 4 * i + u
                            for j in range(EMB // 16):
                                a[j] = a[j] + bufs[nb, row, pl.ds(16 * j, 16)]
                        return tuple(a)

                    acc = lax.fori_loop(0, L // 4, acc_body, acc)
                    r = SEQS * s + t
                    for j in range(EMB // 16):
                        pooled_v[r, pl.ds(16 * j, 16)] = acc[j]

                s2 = s + NBUF

                @pl.when(s2 < NSUP)
                def _(nb=nb, s2=s2):
                    pltpu.async_copy(
                        table_hbm.at[idx_v.at[pl.ds(s2 * ROWS, ROWS)]],
                        bufs.at[nb], sems_g[nb])
            return carry

        lax.fori_loop(0, NSUP // NBUF, outer_body, 0)
        pltpu.sync_copy(pooled_v, out_hbm.at[pl.ds(wid * RPW, RPW)])

    return sc_pool


_sc_pool = _make_sc_pool()


def _linear_body(p_ref, w_ref, b_ref, o_ref):
    pooled = p_ref[...] * (1.0 / L)
    acc = jnp.dot(pooled, w_ref[...].T, preferred_element_type=jnp.float32)
    o_ref[...] = jnp.maximum(acc + b_ref[...], 0.0)


def _linear(pooled_sum, w, b):
    return pl.pallas_call(
        _linear_body,
        out_shape=jax.ShapeDtypeStruct((B, EMB), jnp.float32),
    )(pooled_sum, w, b.reshape(1, EMB))


def kernel(x, table, W, b):
    pooled_sum = _sc_pool(x.astype(jnp.int32), table)
    return _linear(pooled_sum, W, b)
